# baseline (speedup 1.0000x reference)
import jax
import jax.numpy as jnp
from jax import lax
from jax.experimental import pallas as pl
from jax.experimental.pallas import tpu as pltpu
from jax.experimental.pallas import tpu_sc as plsc

N = 10000
E = 160000
D = 256
R = 4096
HALF = 128
NC, NS, LANES = 2, 16, 16


def _I(x):
    return jnp.int32(x)


def _Z():
    return jnp.int32(0)

BN1 = 1000
BR1 = 1024


def _p_matmul_body(h_ref, wrow_ref, re_ref, out_ref):
    hw = h_ref[...] * wrow_ref[...]
    res = lax.dot_general(
        hw, re_ref[...], (((1,), (1,)), ((), ())),
        preferred_element_type=jnp.float32)
    for k in range(BR1 // HALF):
        out_ref[k] = res[:, k * HALF:(k + 1) * HALF]


def _compute_P(h, wrow, rel_embed):
    return pl.pallas_call(
        _p_matmul_body,
        grid=(N // BN1, R // BR1),
        in_specs=[
            pl.BlockSpec((BN1, D), lambda i, j: (i, _Z())),
            pl.BlockSpec((1, D), lambda i, j: (_Z(), _Z())),
            pl.BlockSpec((BR1, D), lambda i, j: (j, _Z())),
        ],
        out_specs=pl.BlockSpec((BR1 // HALF, BN1, HALF),
                               lambda i, j: (j, i, _Z())),
        out_shape=jax.ShapeDtypeStruct((R // HALF, N, HALF), jnp.float32),
    )(h, wrow, rel_embed)


EPT2 = E // (NC * NS)
PAD2 = 5120
GCH2 = 128
NCH2 = PAD2 // GCH2


def _a_gather_body(src_hbm, rel_hbm, pr_hbm, a_hbm,
                   src_v, rel_v, col_v, a_v, rows_a, rows_b, sem_a, sem_b):
    c = lax.axis_index("c")
    s = lax.axis_index("s")
    wid = s * _I(NC) + c
    base = wid * _I(EPT2)
    pltpu.sync_copy(src_hbm.at[pl.ds(base, EPT2)], src_v.at[pl.ds(0, EPT2)])
    pltpu.sync_copy(rel_hbm.at[pl.ds(base, EPT2)], rel_v.at[pl.ds(0, EPT2)])

    def ibody(i, carry):
        sl = pl.ds(i * _I(LANES), LANES)
        rel16 = rel_v[sl]
        row = lax.shift_right_logical(rel16, _I(7)) * _I(N) + src_v[sl]
        src_v[sl] = jnp.minimum(jnp.maximum(row, _I(0)), _I(R // HALF * N - 1))
        col_v[sl] = lax.bitwise_and(rel16, _I(HALF - 1))
        return carry

    lax.fori_loop(_I(0), _I(PAD2 // LANES), ibody, _I(0))

    def g_idx(g):
        return src_v.at[pl.ds(g * _I(GCH2), GCH2)]

    def extract(rows, g):
        for k in range(GCH2 // LANES):
            sl = pl.ds(g * _I(GCH2) + _I(k * LANES), LANES)
            rloc = lax.iota(jnp.int32, LANES) + _I(k * LANES)
            a_v[sl] = plsc.load_gather(rows, [rloc, col_v[sl]])

    pltpu.async_copy(pr_hbm.at[g_idx(_I(0))], rows_a, sem_a)
    pltpu.async_copy(pr_hbm.at[g_idx(_I(1))], rows_b, sem_b)

    def pipe(k, carry):
        ga = k * _I(2)
        gb = ga + _I(1)
        pltpu.make_async_copy(pr_hbm.at[g_idx(ga)], rows_a, sem_a).wait()
        extract(rows_a, ga)

        @pl.when(k < _I(NCH2 // 2 - 1))
        def _():
            pltpu.async_copy(pr_hbm.at[g_idx(ga + _I(2))], rows_a, sem_a)

        pltpu.make_async_copy(pr_hbm.at[g_idx(gb)], rows_b, sem_b).wait()
        extract(rows_b, gb)

        @pl.when(k < _I(NCH2 // 2 - 1))
        def _():
            pltpu.async_copy(pr_hbm.at[g_idx(gb + _I(2))], rows_b, sem_b)

        return carry

    lax.fori_loop(_I(0), _I(NCH2 // 2), pipe, _I(0))
    pltpu.sync_copy(a_v.at[pl.ds(0, EPT2)], a_hbm.at[pl.ds(base, EPT2)])


def _a_gather(src, rel, pr):
    mesh = plsc.VectorSubcoreMesh(core_axis_name="c", subcore_axis_name="s")
    f = pl.kernel(
        _a_gather_body,
        out_type=jax.ShapeDtypeStruct((E,), jnp.float32),
        mesh=mesh,
        compiler_params=pltpu.CompilerParams(needs_layout_passes=False),
        scratch_types=[
            pltpu.VMEM((PAD2,), jnp.int32),
            pltpu.VMEM((PAD2,), jnp.int32),
            pltpu.VMEM((PAD2,), jnp.int32),
            pltpu.VMEM((PAD2,), jnp.float32),
            pltpu.VMEM((GCH2, HALF), jnp.float32),
            pltpu.VMEM((GCH2, HALF), jnp.float32),
            pltpu.SemaphoreType.DMA,
            pltpu.SemaphoreType.DMA,
        ],
    )
    return f(src, rel, pr)


def _softmax_body(a_ref, t_ref, s_ref):
    x = a_ref[...]
    t = jnp.exp(x - jnp.max(x))
    t_ref[...] = t
    s_ref[...] = jnp.sum(t).reshape(1, 1)


def _softmax(a2):
    return pl.pallas_call(
        _softmax_body,
        out_shape=(jax.ShapeDtypeStruct(a2.shape, jnp.float32),
                   jax.ShapeDtypeStruct((1, 1), jnp.float32)),
    )(a2)


EPT3 = E // NS
CH3 = 80
NCH3 = EPT3 // CH3
STRIPE = N // NS


def _scatter_body(src_hbm, dst2_hbm, t_hbm, h2_hbm, acc_hbm, deg_hbm,
                  idx_v, dst_v, t_a, t_b, rows_a, rows_b, ones_v, zdeg_v,
                  acc_sh, deg_sh, gsem_a, gsem_b, ssem_a, ssem_b):
    c = lax.axis_index("c")
    s = lax.axis_index("s")
    base = s * _I(EPT3)
    pltpu.sync_copy(src_hbm.at[pl.ds(base, EPT3)], idx_v)
    pltpu.sync_copy(dst2_hbm.at[s], dst_v)

    def ibody(i, carry):
        sl = pl.ds(i * _I(LANES), LANES)
        idx_v[sl] = idx_v[sl] * _I(2) + jnp.full((LANES,), c, jnp.int32)
        return carry

    lax.fori_loop(_I(0), _I(EPT3 // LANES), ibody, _I(0))

    for k in range(CH3 // LANES):
        ones_v[pl.ds(k * LANES, LANES)] = jnp.ones((LANES,), jnp.float32)

    def zbody(e, carry):
        for j in range(HALF // LANES):
            rows_a[e, pl.ds(j * LANES, LANES)] = jnp.zeros((LANES,), jnp.float32)
        return carry

    lax.fori_loop(_I(0), _I(CH3), zbody, _I(0))
    for k in range(40):
        zdeg_v[pl.ds(k * LANES, LANES)] = jnp.zeros((LANES,), jnp.float32)

    srow = s * _I(640)

    @pl.when(s < NS - 1)
    def _():
        for k in range(640 // CH3):
            pltpu.sync_copy(rows_a, acc_sh.at[pl.ds(srow + k * CH3, CH3)])

    @pl.when(s == NS - 1)
    def _():
        for k in range(400 // CH3):
            pltpu.sync_copy(rows_a, acc_sh.at[pl.ds(9600 + k * CH3, CH3)])

    @pl.when(s == 0)
    def _():
        for k in range(N // 640):
            pltpu.sync_copy(zdeg_v, deg_sh.at[pl.ds(k * 640, 640)])
        pltpu.sync_copy(zdeg_v.at[pl.ds(0, N - (N // 640) * 640)],
                        deg_sh.at[pl.ds((N // 640) * 640, N - (N // 640) * 640)])

    def g_idx(g):
        return idx_v.at[pl.ds(g * _I(CH3), CH3)]

    def t_slice(g):
        return t_hbm.at[pl.ds(base + g * _I(CH3), CH3)]

    def issue(g, buf, tbuf, gsem):
        pltpu.async_copy(h2_hbm.at[g_idx(g)], buf, gsem)
        pltpu.async_copy(t_slice(g), tbuf, gsem)

    def scale(buf, tbuf):
        def ebody(k, carry2):
            for u in range(4):
                e = k * _I(4) + _I(u)
                tsplat = plsc.load_gather(
                    tbuf, [jnp.full((LANES,), e, jnp.int32)])
                for j in range(HALF // LANES):
                    sl = pl.ds(j * LANES, LANES)
                    buf[e, sl] = buf[e, sl] * tsplat
            return carry2

        lax.fori_loop(_I(0), _I(CH3 // 4), ebody, _I(0))

    def process(buf, tbuf, g, gsem, ssem):
        pltpu.make_async_copy(h2_hbm.at[g_idx(g)], buf, gsem).wait()
        pltpu.make_async_copy(t_slice(g), tbuf, gsem).wait()
        scale(buf, tbuf)
        pltpu.async_copy(buf, acc_sh.at[dst_v.at[g]], ssem, add=True)

        @pl.when(c == 0)
        def _():
            pltpu.async_copy(ones_v, deg_sh.at[dst_v.at[g]], ssem, add=True)

    def drain_scatter(buf, g, ssem):
        pltpu.make_async_copy(buf, acc_sh.at[dst_v.at[g]], ssem).wait()

        @pl.when(c == 0)
        def _():
            pltpu.make_async_copy(ones_v, deg_sh.at[dst_v.at[g]], ssem).wait()

    issue(_I(0), rows_a, t_a, gsem_a)
    issue(_I(1), rows_b, t_b, gsem_b)
    plsc.subcore_barrier()

    def pipe(k, carry):
        ga = k * _I(2)
        gb = ga + _I(1)
        process(rows_a, t_a, ga, gsem_a, ssem_a)
        process(rows_b, t_b, gb, gsem_b, ssem_b)
        drain_scatter(rows_a, ga, ssem_a)
        issue(ga + _I(2), rows_a, t_a, gsem_a)
        drain_scatter(rows_b, gb, ssem_b)

        @pl.when(k < _I(NCH3 // 2 - 1))
        def _():
            issue(gb + _I(2), rows_b, t_b, gsem_b)

        return carry

    lax.fori_loop(_I(0), _I(NCH3 // 2), pipe, _I(0))
    gl = _I(NCH3 - 1)
    pltpu.make_async_copy(h2_hbm.at[g_idx(gl)], rows_a, gsem_a).wait()
    pltpu.make_async_copy(t_slice(gl), t_a, gsem_a).wait()
    scale(rows_a, t_a)
    pltpu.sync_copy(rows_a, acc_sh.at[dst_v.at[gl]], add=True)

    @pl.when(c == 0)
    def _():
        pltpu.sync_copy(ones_v, deg_sh.at[dst_v.at[gl]], add=True)

    plsc.subcore_barrier()

    @pl.when(s < NS - 1)
    def _():
        pltpu.sync_copy(acc_sh.at[pl.ds(srow, 640)],
                        acc_hbm.at[c, pl.ds(srow, 640)])

    @pl.when(s == NS - 1)
    def _():
        pltpu.sync_copy(acc_sh.at[pl.ds(9600, 400)],
                        acc_hbm.at[c, pl.ds(9600, 400)])

    @pl.when(jnp.logical_and(c == 0, s == 0))
    def _():
        pltpu.sync_copy(deg_sh, deg_hbm)


def _scatter(src, dst2, t, h2):
    mesh = plsc.VectorSubcoreMesh(core_axis_name="c", subcore_axis_name="s")
    f = pl.kernel(
        _scatter_body,
        out_type=(jax.ShapeDtypeStruct((NC, N, HALF), jnp.float32),
                  jax.ShapeDtypeStruct((N,), jnp.float32)),
        mesh=mesh,
        compiler_params=pltpu.CompilerParams(needs_layout_passes=False),
        scratch_types=[
            pltpu.VMEM((EPT3,), jnp.int32),
            pltpu.VMEM((NCH3, CH3), jnp.int32),
            pltpu.VMEM((CH3,), jnp.float32),
            pltpu.VMEM((CH3,), jnp.float32),
            pltpu.VMEM((CH3, HALF), jnp.float32),
            pltpu.VMEM((CH3, HALF), jnp.float32),
            pltpu.VMEM((CH3,), jnp.float32),
            pltpu.VMEM((640,), jnp.float32),
            pltpu.VMEM_SHARED((N, HALF), jnp.float32),
            pltpu.VMEM_SHARED((N,), jnp.float32),
            pltpu.SemaphoreType.DMA,
            pltpu.SemaphoreType.DMA,
            pltpu.SemaphoreType.DMA,
            pltpu.SemaphoreType.DMA,
        ],
    )
    return f(src, dst2, t, h2)


BN4 = 1000


def _final_body(h_ref, a0_ref, a1_ref, deg_ref, s_ref, ww_ref, o_ref):
    inv = 1.0 / (s_ref[...] * jnp.maximum(deg_ref[...], 1.0))
    y = lax.dot_general(a0_ref[...] * inv, ww_ref[0:HALF, :],
                        (((1,), (0,)), ((), ())),
                        preferred_element_type=jnp.float32)
    y = y + lax.dot_general(a1_ref[...] * inv, ww_ref[HALF:D, :],
                            (((1,), (0,)), ((), ())),
                            preferred_element_type=jnp.float32)
    o_ref[...] = h_ref[...] + y


def _final(h, a0, a1, deg2, S, W_w):
    return pl.pallas_call(
        _final_body,
        grid=(N // BN4,),
        in_specs=[
            pl.BlockSpec((BN4, D), lambda i: (i, _Z())),
            pl.BlockSpec((BN4, HALF), lambda i: (i, _Z())),
            pl.BlockSpec((BN4, HALF), lambda i: (i, _Z())),
            pl.BlockSpec((BN4, 1), lambda i: (i, _Z())),
            pl.BlockSpec((1, 1), lambda i: (_Z(), _Z())),
            pl.BlockSpec((D, D), lambda i: (_Z(), _Z())),
        ],
        out_specs=pl.BlockSpec((BN4, D), lambda i: (i, _Z())),
        out_shape=jax.ShapeDtypeStruct((N, D), jnp.float32),
    )(h, a0, a1, deg2, S, W_w)


def kernel(h, edge_index, rel_ids, rel_embed, W_w, W_att, b_att):
    del b_att
    h = h.astype(jnp.float32)
    src = edge_index[0].astype(jnp.int32)
    dst = edge_index[1].astype(jnp.int32)
    rel = rel_ids.astype(jnp.int32)
    wrow = W_att.astype(jnp.float32).reshape(1, D)

    P3 = _compute_P(h, wrow, rel_embed.astype(jnp.float32))
    a = _a_gather(src, rel, P3.reshape(R // HALF * N, HALF))
    t2, S = _softmax(a.reshape(E // HALF, HALF))
    t = t2.reshape(E)

    h2 = h.reshape(2 * N, HALF)
    dst2 = dst.reshape(NS, NCH3, CH3)
    accbuf, deg = _scatter(src, dst2, t, h2)

    out = _final(h, accbuf[0], accbuf[1], deg.reshape(N, 1),
                 S, W_w.astype(jnp.float32))
    return out.astype(jnp.float64)


# speedup vs baseline: 1.2327x; 1.2327x over previous
import jax
import jax.numpy as jnp
from jax import lax
from jax.experimental import pallas as pl
from jax.experimental.pallas import tpu as pltpu
from jax.experimental.pallas import tpu_sc as plsc

N = 10000
E = 160000
D = 256
R = 4096
HALF = 128
NC, NS, LANES = 2, 16, 16


def _I(x):
    return jnp.int32(x)


def _Z():
    return jnp.int32(0)

BN1 = 1000
BR1 = 1024


def _p_matmul_body(h_ref, wrow_ref, re_ref, out_ref):
    hw = h_ref[...] * wrow_ref[...]
    res = lax.dot_general(
        hw, re_ref[...], (((1,), (1,)), ((), ())),
        preferred_element_type=jnp.float32)
    for k in range(BR1 // HALF):
        out_ref[k] = res[:, k * HALF:(k + 1) * HALF]


def _compute_P(h, wrow, rel_embed):
    return pl.pallas_call(
        _p_matmul_body,
        grid=(N // BN1, R // BR1),
        in_specs=[
            pl.BlockSpec((BN1, D), lambda i, j: (i, _Z())),
            pl.BlockSpec((1, D), lambda i, j: (_Z(), _Z())),
            pl.BlockSpec((BR1, D), lambda i, j: (j, _Z())),
        ],
        out_specs=pl.BlockSpec((BR1 // HALF, BN1, HALF),
                               lambda i, j: (j, i, _Z())),
        out_shape=jax.ShapeDtypeStruct((R // HALF, N, HALF), jnp.float32),
    )(h, wrow, rel_embed)


EPT2 = E // (NC * NS)
PAD2 = 5120
GCH2 = 128
NCH2 = PAD2 // GCH2


def _a_gather_body(src_hbm, rel_hbm, pflat_hbm, a_hbm,
                   src_v, rel_v, idx_v, a_v, sem):
    c = lax.axis_index("c")
    s = lax.axis_index("s")
    wid = s * _I(NC) + c
    base = wid * _I(EPT2)
    pltpu.sync_copy(src_hbm.at[pl.ds(base, EPT2)], src_v.at[pl.ds(0, EPT2)])
    pltpu.sync_copy(rel_hbm.at[pl.ds(base, EPT2)], rel_v.at[pl.ds(0, EPT2)])

    def ibody(i, carry):
        sl = pl.ds(i * _I(LANES), LANES)
        rel16 = rel_v[sl]
        flat = (lax.shift_right_logical(rel16, _I(7)) * _I(N * HALF)
                + src_v[sl] * _I(HALF)
                + lax.bitwise_and(rel16, _I(HALF - 1)))
        idx_v[sl] = jnp.minimum(jnp.maximum(flat, _I(0)), _I(N * R - 1))
        return carry

    lax.fori_loop(_I(0), _I(PAD2 // LANES), ibody, _I(0))

    copies = []
    for g in range(NCH2):
        sl = pl.ds(g * GCH2, GCH2)
        copies.append(pltpu.async_copy(
            pflat_hbm.at[idx_v.at[sl]], a_v.at[sl], sem))
    for cp in copies:
        cp.wait()
    pltpu.sync_copy(a_v.at[pl.ds(0, EPT2)], a_hbm.at[pl.ds(base, EPT2)])


def _a_gather(src, rel, pflat):
    mesh = plsc.VectorSubcoreMesh(core_axis_name="c", subcore_axis_name="s")
    f = pl.kernel(
        _a_gather_body,
        out_type=jax.ShapeDtypeStruct((E,), jnp.float32),
        mesh=mesh,
        scratch_types=[
            pltpu.VMEM((PAD2,), jnp.int32),
            pltpu.VMEM((PAD2,), jnp.int32),
            pltpu.VMEM((PAD2,), jnp.int32),
            pltpu.VMEM((PAD2,), jnp.float32),
            pltpu.SemaphoreType.DMA,
        ],
    )
    return f(src, rel, pflat)


def _softmax_body(a_ref, t_ref, s_ref):
    x = a_ref[...]
    t = jnp.exp(x - jnp.max(x))
    t_ref[...] = t
    s_ref[...] = jnp.sum(t).reshape(1, 1)


def _softmax(a2):
    return pl.pallas_call(
        _softmax_body,
        out_shape=(jax.ShapeDtypeStruct(a2.shape, jnp.float32),
                   jax.ShapeDtypeStruct((1, 1), jnp.float32)),
    )(a2)


EPT3 = E // NS
CH3 = 80
NCH3 = EPT3 // CH3
STRIPE = N // NS


def _scatter_body(src_hbm, dst2_hbm, t_hbm, h2_hbm, acc_hbm, deg_hbm,
                  idx_v, dst_v, t_a, t_b, rows_a, rows_b, ones_v, zdeg_v,
                  acc_sh, deg_sh, gsem_a, gsem_b, ssem_a, ssem_b):
    c = lax.axis_index("c")
    s = lax.axis_index("s")
    base = s * _I(EPT3)
    pltpu.sync_copy(src_hbm.at[pl.ds(base, EPT3)], idx_v)
    pltpu.sync_copy(dst2_hbm.at[s], dst_v)

    def ibody(i, carry):
        sl = pl.ds(i * _I(LANES), LANES)
        idx_v[sl] = idx_v[sl] * _I(2) + jnp.full((LANES,), c, jnp.int32)
        return carry

    lax.fori_loop(_I(0), _I(EPT3 // LANES), ibody, _I(0))

    for k in range(CH3 // LANES):
        ones_v[pl.ds(k * LANES, LANES)] = jnp.ones((LANES,), jnp.float32)

    def zbody(e, carry):
        for j in range(HALF // LANES):
            rows_a[e, pl.ds(j * LANES, LANES)] = jnp.zeros((LANES,), jnp.float32)
        return carry

    lax.fori_loop(_I(0), _I(CH3), zbody, _I(0))
    for k in range(40):
        zdeg_v[pl.ds(k * LANES, LANES)] = jnp.zeros((LANES,), jnp.float32)

    srow = s * _I(640)

    @pl.when(s < NS - 1)
    def _():
        for k in range(640 // CH3):
            pltpu.sync_copy(rows_a, acc_sh.at[pl.ds(srow + k * CH3, CH3)])

    @pl.when(s == NS - 1)
    def _():
        for k in range(400 // CH3):
            pltpu.sync_copy(rows_a, acc_sh.at[pl.ds(9600 + k * CH3, CH3)])

    @pl.when(s == 0)
    def _():
        for k in range(N // 640):
            pltpu.sync_copy(zdeg_v, deg_sh.at[pl.ds(k * 640, 640)])
        pltpu.sync_copy(zdeg_v.at[pl.ds(0, N - (N // 640) * 640)],
                        deg_sh.at[pl.ds((N // 640) * 640, N - (N // 640) * 640)])

    def g_idx(g):
        return idx_v.at[pl.ds(g * _I(CH3), CH3)]

    def t_slice(g):
        return t_hbm.at[pl.ds(base + g * _I(CH3), CH3)]

    def issue(g, buf, tbuf, gsem):
        pltpu.async_copy(h2_hbm.at[g_idx(g)], buf, gsem)
        pltpu.async_copy(t_slice(g), tbuf, gsem)

    def scale(buf, tbuf):
        def ebody(k, carry2):
            for u in range(4):
                e = k * _I(4) + _I(u)
                tsplat = plsc.load_gather(
                    tbuf, [jnp.full((LANES,), e, jnp.int32)])
                for j in range(HALF // LANES):
                    sl = pl.ds(j * LANES, LANES)
                    buf[e, sl] = buf[e, sl] * tsplat
            return carry2

        lax.fori_loop(_I(0), _I(CH3 // 4), ebody, _I(0))

    def process(buf, tbuf, g, gsem, ssem):
        pltpu.make_async_copy(h2_hbm.at[g_idx(g)], buf, gsem).wait()
        pltpu.make_async_copy(t_slice(g), tbuf, gsem).wait()
        scale(buf, tbuf)
        pltpu.async_copy(buf, acc_sh.at[dst_v.at[g]], ssem, add=True)

        @pl.when(c == 0)
        def _():
            pltpu.async_copy(ones_v, deg_sh.at[dst_v.at[g]], ssem, add=True)

    def drain_scatter(buf, g, ssem):
        pltpu.make_async_copy(buf, acc_sh.at[dst_v.at[g]], ssem).wait()

        @pl.when(c == 0)
        def _():
            pltpu.make_async_copy(ones_v, deg_sh.at[dst_v.at[g]], ssem).wait()

    issue(_I(0), rows_a, t_a, gsem_a)
    issue(_I(1), rows_b, t_b, gsem_b)
    plsc.subcore_barrier()

    def pipe(k, carry):
        ga = k * _I(2)
        gb = ga + _I(1)
        process(rows_a, t_a, ga, gsem_a, ssem_a)
        process(rows_b, t_b, gb, gsem_b, ssem_b)
        drain_scatter(rows_a, ga, ssem_a)
        issue(ga + _I(2), rows_a, t_a, gsem_a)
        drain_scatter(rows_b, gb, ssem_b)

        @pl.when(k < _I(NCH3 // 2 - 1))
        def _():
            issue(gb + _I(2), rows_b, t_b, gsem_b)

        return carry

    lax.fori_loop(_I(0), _I(NCH3 // 2), pipe, _I(0))
    gl = _I(NCH3 - 1)
    pltpu.make_async_copy(h2_hbm.at[g_idx(gl)], rows_a, gsem_a).wait()
    pltpu.make_async_copy(t_slice(gl), t_a, gsem_a).wait()
    scale(rows_a, t_a)
    pltpu.sync_copy(rows_a, acc_sh.at[dst_v.at[gl]], add=True)

    @pl.when(c == 0)
    def _():
        pltpu.sync_copy(ones_v, deg_sh.at[dst_v.at[gl]], add=True)

    plsc.subcore_barrier()

    @pl.when(s < NS - 1)
    def _():
        pltpu.sync_copy(acc_sh.at[pl.ds(srow, 640)],
                        acc_hbm.at[c, pl.ds(srow, 640)])

    @pl.when(s == NS - 1)
    def _():
        pltpu.sync_copy(acc_sh.at[pl.ds(9600, 400)],
                        acc_hbm.at[c, pl.ds(9600, 400)])

    @pl.when(jnp.logical_and(c == 0, s == 0))
    def _():
        pltpu.sync_copy(deg_sh, deg_hbm)


def _scatter(src, dst2, t, h2):
    mesh = plsc.VectorSubcoreMesh(core_axis_name="c", subcore_axis_name="s")
    f = pl.kernel(
        _scatter_body,
        out_type=(jax.ShapeDtypeStruct((NC, N, HALF), jnp.float32),
                  jax.ShapeDtypeStruct((N,), jnp.float32)),
        mesh=mesh,
        compiler_params=pltpu.CompilerParams(needs_layout_passes=False),
        scratch_types=[
            pltpu.VMEM((EPT3,), jnp.int32),
            pltpu.VMEM((NCH3, CH3), jnp.int32),
            pltpu.VMEM((CH3,), jnp.float32),
            pltpu.VMEM((CH3,), jnp.float32),
            pltpu.VMEM((CH3, HALF), jnp.float32),
            pltpu.VMEM((CH3, HALF), jnp.float32),
            pltpu.VMEM((CH3,), jnp.float32),
            pltpu.VMEM((640,), jnp.float32),
            pltpu.VMEM_SHARED((N, HALF), jnp.float32),
            pltpu.VMEM_SHARED((N,), jnp.float32),
            pltpu.SemaphoreType.DMA,
            pltpu.SemaphoreType.DMA,
            pltpu.SemaphoreType.DMA,
            pltpu.SemaphoreType.DMA,
        ],
    )
    return f(src, dst2, t, h2)


BN4 = 1000


def _final_body(h_ref, a0_ref, a1_ref, deg_ref, s_ref, ww_ref, o_ref):
    inv = 1.0 / (s_ref[...] * jnp.maximum(deg_ref[...], 1.0))
    y = lax.dot_general(a0_ref[...] * inv, ww_ref[0:HALF, :],
                        (((1,), (0,)), ((), ())),
                        preferred_element_type=jnp.float32)
    y = y + lax.dot_general(a1_ref[...] * inv, ww_ref[HALF:D, :],
                            (((1,), (0,)), ((), ())),
                            preferred_element_type=jnp.float32)
    o_ref[...] = h_ref[...] + y


def _final(h, a0, a1, deg2, S, W_w):
    return pl.pallas_call(
        _final_body,
        grid=(N // BN4,),
        in_specs=[
            pl.BlockSpec((BN4, D), lambda i: (i, _Z())),
            pl.BlockSpec((BN4, HALF), lambda i: (i, _Z())),
            pl.BlockSpec((BN4, HALF), lambda i: (i, _Z())),
            pl.BlockSpec((BN4, 1), lambda i: (i, _Z())),
            pl.BlockSpec((1, 1), lambda i: (_Z(), _Z())),
            pl.BlockSpec((D, D), lambda i: (_Z(), _Z())),
        ],
        out_specs=pl.BlockSpec((BN4, D), lambda i: (i, _Z())),
        out_shape=jax.ShapeDtypeStruct((N, D), jnp.float32),
    )(h, a0, a1, deg2, S, W_w)


def kernel(h, edge_index, rel_ids, rel_embed, W_w, W_att, b_att):
    del b_att
    h = h.astype(jnp.float32)
    src = edge_index[0].astype(jnp.int32)
    dst = edge_index[1].astype(jnp.int32)
    rel = rel_ids.astype(jnp.int32)
    wrow = W_att.astype(jnp.float32).reshape(1, D)

    P3 = _compute_P(h, wrow, rel_embed.astype(jnp.float32))
    a = _a_gather(src, rel, P3.reshape(N * R))
    t2, S = _softmax(a.reshape(E // HALF, HALF))
    t = t2.reshape(E)

    h2 = h.reshape(2 * N, HALF)
    dst2 = dst.reshape(NS, NCH3, CH3)
    accbuf, deg = _scatter(src, dst2, t, h2)

    out = _final(h, accbuf[0], accbuf[1], deg.reshape(N, 1),
                 S, W_w.astype(jnp.float32))
    return out.astype(jnp.float64)
